# baseline (speedup 1.0000x reference)
import functools

import jax
import jax.numpy as jnp
from jax import lax
from jax.experimental import pallas as pl
from jax.experimental.pallas import tpu as pltpu
from jax.experimental.pallas import tpu_sc as plsc

MAXLEN_ = 200
EMBED_ = 64
BATCH_ = 4096
LANES_ = 16
SPLIT_ = 96
REST_ = MAXLEN_ - SPLIT_
IVLEN_ = 216
NSHIFT_ = 13
OBW_ = 3200
TPR_ = 50


def kernel(x, token_table, pos_table):
    info = plsc.get_sparse_core_info()
    nc, ns = info.num_cores, info.num_subcores
    nw = nc * ns
    rows_per_w = BATCH_ // nw
    nch = rows_per_w // 2

    mesh = plsc.VectorSubcoreMesh(core_axis_name="c", subcore_axis_name="s")

    scratch = (
        [pltpu.VMEM((IVLEN_,), jnp.int32) for _ in range(2)]
        + [pltpu.VMEM((IVLEN_,), jnp.int32) for _ in range(2)]
        + [pltpu.SMEM((MAXLEN_,), jnp.int32) for _ in range(2)]
        + [pltpu.VMEM((MAXLEN_, 128), jnp.float32) for _ in range(2)]
        + [pltpu.VMEM((8, OBW_), jnp.float32) for _ in range(2)]
        + [pltpu.VMEM((8, OBW_), jnp.float32)]
        + [pltpu.SemaphoreType.DMA for _ in range(8)]
    )

    @functools.partial(
        pl.kernel,
        mesh=mesh,
        out_type=jax.ShapeDtypeStruct((BATCH_ // 2, 8, OBW_), jnp.float32),
        scratch_types=scratch,
    )
    def emb_kernel(xf_hbm, tt_hbm, pos_hbm, out_hbm, *refs):
        iv = refs[0:2]
        gv = refs[2:4]
        sm = refs[4:6]
        buf = refs[6:8]
        ob = refs[8:10]
        pv = refs[10]
        ivsem = refs[11:13]
        ssem = refs[13:15]
        gsem = refs[15:17]
        osem = refs[17:19]

        wid = lax.axis_index("s") * nc + lax.axis_index("c")
        row0 = wid * rows_per_w
        g0 = wid * nch

        def issue_idx(s, b):
            off = (row0 + s) * MAXLEN_
            pltpu.async_copy(
                xf_hbm.at[pl.ds(off, MAXLEN_)], iv[b].at[pl.ds(0, MAXLEN_)], ivsem[b]
            )

        def wait_idx(b):
            pltpu.make_async_copy(
                xf_hbm.at[pl.ds(0, MAXLEN_)], iv[b].at[pl.ds(0, MAXLEN_)], ivsem[b]
            ).wait()

        def issue_sm(b):
            pltpu.async_copy(iv[b].at[pl.ds(0, MAXLEN_)], sm[b], ssem[b])

        def wait_sm(b):
            pltpu.make_async_copy(
                iv[b].at[pl.ds(0, MAXLEN_)], sm[b], ssem[b]
            ).wait()

        def shift(b):
            for k in range(NSHIFT_):
                sl = pl.ds(k * LANES_, LANES_)
                gv[b][sl] = lax.shift_right_logical(iv[b][sl], 1)

        def issue_gather(b):
            pltpu.async_copy(
                tt_hbm.at[gv[b].at[pl.ds(0, SPLIT_)]], buf[b].at[pl.ds(0, SPLIT_)],
                gsem[b],
            )
            pltpu.async_copy(
                tt_hbm.at[gv[b].at[pl.ds(SPLIT_, REST_)]],
                buf[b].at[pl.ds(SPLIT_, REST_)],
                gsem[b],
            )

        def wait_gather(b):
            pltpu.make_async_copy(
                tt_hbm.at[gv[b].at[pl.ds(0, SPLIT_)]], buf[b].at[pl.ds(0, SPLIT_)],
                gsem[b],
            ).wait()
            pltpu.make_async_copy(
                tt_hbm.at[gv[b].at[pl.ds(SPLIT_, REST_)]],
                buf[b].at[pl.ds(SPLIT_, REST_)],
                gsem[b],
            ).wait()

        def select_add(b, os_):
            r0 = 4 * b
            for rr in range(4):
                buf_b, sm_b, ob_o = buf[b], iv[b], ob[os_]

                def token(t, carry):
                    i = rr * TPR_ + t
                    tv = sm_b[pl.ds(i, LANES_)]
                    half = (tv[0] & 1) * EMBED_
                    c0 = t * EMBED_
                    for j in range(EMBED_ // LANES_):
                        ob_o[r0 + rr, pl.ds(c0 + j * LANES_, LANES_)] = (
                            buf_b[i, pl.ds(half + j * LANES_, LANES_)]
                            + pv[r0 + rr, pl.ds(c0 + j * LANES_, LANES_)]
                        )
                    return carry

                lax.fori_loop(0, TPR_, token, 0)

        def issue_out(g, os_):
            pltpu.async_copy(ob[os_], out_hbm.at[g0 + g], osem[os_])

        def wait_out(os_):
            pltpu.make_async_copy(ob[os_], out_hbm.at[g0], osem[os_]).wait()

        def block(s, u, *, with_out_wait, with_idx, with_gather):
            b = u % 2
            os_ = u // 2
            if with_gather:
                wait_idx(1 - b)
                shift(1 - b)
                issue_gather(1 - b)
            wait_gather(b)
            if b == 0 and with_out_wait:
                wait_out(os_)
            select_add(b, os_)
            if b == 1:
                issue_out(s // 2, os_)
            if with_idx:
                issue_idx(s + 2, b)

        pltpu.sync_copy(pos_hbm, pv)

        issue_idx(0, 0)
        issue_idx(1, 1)
        wait_idx(0)
        shift(0)
        issue_gather(0)

        block(0, 0, with_out_wait=False, with_idx=True, with_gather=True)
        block(1, 1, with_out_wait=False, with_idx=True, with_gather=True)
        block(2, 2, with_out_wait=False, with_idx=True, with_gather=True)
        block(3, 3, with_out_wait=False, with_idx=True, with_gather=True)

        def group(gg, carry):
            s0 = gg * 4
            for u in range(4):
                block(s0 + u, u, with_out_wait=True, with_idx=True, with_gather=True)
            return carry

        lax.fori_loop(1, rows_per_w // 4 - 1, group, 0)

        s0 = rows_per_w - 4
        block(s0 + 0, 0, with_out_wait=True, with_idx=True, with_gather=True)
        block(s0 + 1, 1, with_out_wait=True, with_idx=True, with_gather=True)
        block(s0 + 2, 2, with_out_wait=True, with_idx=False, with_gather=True)
        block(s0 + 3, 3, with_out_wait=True, with_idx=False, with_gather=False)
        wait_out(0)
        wait_out(1)

    xf = x.astype(jnp.int32).reshape(-1)
    tt2 = token_table.reshape(-1, 128)
    pos8 = jnp.tile(pos_table.reshape(-1), 2).reshape(8, OBW_)
    out = emb_kernel(xf, tt2, pos8)
    return out.reshape(BATCH_, MAXLEN_, EMBED_)


# speedup vs baseline: 2.1284x; 2.1284x over previous
import functools

import jax
import jax.numpy as jnp
from jax import lax
from jax.experimental import pallas as pl
from jax.experimental.pallas import tpu as pltpu
from jax.experimental.pallas import tpu_sc as plsc

MAXLEN_ = 200
EMBED_ = 64
BATCH_ = 4096
LANES_ = 16
SPLIT_ = 96
REST_ = MAXLEN_ - SPLIT_
NB_ = 4


def kernel(x, token_table, pos_table):
    info = plsc.get_sparse_core_info()
    nc, ns = info.num_cores, info.num_subcores
    nw = nc * ns
    rows_per_w = BATCH_ // nw
    nch = rows_per_w

    mesh = plsc.VectorSubcoreMesh(core_axis_name="c", subcore_axis_name="s")

    scratch = (
        [pltpu.VMEM((SPLIT_,), jnp.int32) for _ in range(NB_)]
        + [pltpu.VMEM((REST_,), jnp.int32) for _ in range(NB_)]
        + [pltpu.VMEM((MAXLEN_, EMBED_), jnp.float32) for _ in range(NB_)]
        + [pltpu.VMEM((MAXLEN_, EMBED_), jnp.float32)]
        + [pltpu.SemaphoreType.DMA for _ in range(3 * NB_)]
    )

    @functools.partial(
        pl.kernel,
        mesh=mesh,
        compiler_params=pltpu.CompilerParams(use_tc_tiling_on_sc=False),
        out_type=jax.ShapeDtypeStruct((BATCH_, MAXLEN_, 128), jnp.float32),
        scratch_types=scratch,
    )
    def emb_kernel(x_hbm, tt_hbm, pt_hbm, out_hbm, *refs):
        ia = refs[0:NB_]
        ib = refs[NB_ : 2 * NB_]
        tok = refs[2 * NB_ : 3 * NB_]
        pos = refs[3 * NB_]
        isem = refs[3 * NB_ + 1 : 3 * NB_ + 1 + NB_]
        gsem = refs[3 * NB_ + 1 + NB_ : 3 * NB_ + 1 + 2 * NB_]
        osem = refs[3 * NB_ + 1 + 2 * NB_ : 3 * NB_ + 1 + 3 * NB_]

        wid = lax.axis_index("s") * nc + lax.axis_index("c")
        base = wid * rows_per_w

        def issue_idx(c, b):
            off = (base + c) * MAXLEN_
            pltpu.async_copy(x_hbm.at[pl.ds(off, SPLIT_)], ia[b], isem[b])
            pltpu.async_copy(x_hbm.at[pl.ds(off + SPLIT_, REST_)], ib[b], isem[b])

        def wait_idx(b):
            pltpu.make_async_copy(x_hbm.at[pl.ds(0, SPLIT_)], ia[b], isem[b]).wait()
            pltpu.make_async_copy(x_hbm.at[pl.ds(0, REST_)], ib[b], isem[b]).wait()

        def issue_gather(b):
            pltpu.async_copy(tt_hbm.at[ia[b]], tok[b].at[pl.ds(0, SPLIT_)], gsem[b])
            pltpu.async_copy(
                tt_hbm.at[ib[b]], tok[b].at[pl.ds(SPLIT_, REST_)], gsem[b]
            )

        def wait_gather(b):
            pltpu.make_async_copy(
                tt_hbm.at[ia[b]], tok[b].at[pl.ds(0, SPLIT_)], gsem[b]
            ).wait()
            pltpu.make_async_copy(
                tt_hbm.at[ib[b]], tok[b].at[pl.ds(SPLIT_, REST_)], gsem[b]
            ).wait()

        def issue_out(c, b):
            pltpu.async_copy(
                tok[b], out_hbm.at[base + c, :, pl.ds(0, EMBED_)], osem[b]
            )

        def wait_out(b):
            pltpu.make_async_copy(
                tok[b], out_hbm.at[base, :, pl.ds(0, EMBED_)], osem[b]
            ).wait()

        def add(b):
            tok_b = tok[b]

            def add_row(i, carry):
                for j in range(EMBED_ // LANES_):
                    sl = pl.ds(j * LANES_, LANES_)
                    tok_b[i, sl] = tok_b[i, sl] + pos[i, sl]
                return carry

            lax.fori_loop(0, MAXLEN_, add_row, 0)

        def step(c, b, *, with_out_wait):
            issue_idx(c + 2, (b + 2) % NB_)
            wait_idx((b + 1) % NB_)
            if with_out_wait:
                wait_out((b + 1) % NB_)
            issue_gather((b + 1) % NB_)
            wait_gather(b)
            add(b)
            issue_out(c, b)

        pltpu.sync_copy(pt_hbm, pos)

        issue_idx(0, 0)
        issue_idx(1, 1)
        wait_idx(0)
        issue_gather(0)
        step(0, 0, with_out_wait=False)
        step(1, 1, with_out_wait=False)
        step(2, 2, with_out_wait=False)
        step(3, 3, with_out_wait=True)

        def group(g, carry):
            c0 = g * NB_
            for b in range(NB_):
                step(c0 + b, b, with_out_wait=True)
            return carry

        lax.fori_loop(1, nch // NB_ - 1, group, 0)

        c0 = nch - NB_
        issue_idx(c0 + 2, 2)
        wait_idx(1)
        wait_out(1)
        issue_gather(1)
        wait_gather(0)
        add(0)
        issue_out(c0, 0)
        issue_idx(c0 + 3, 3)
        wait_idx(2)
        wait_out(2)
        issue_gather(2)
        wait_gather(1)
        add(1)
        issue_out(c0 + 1, 1)
        wait_idx(3)
        wait_out(3)
        issue_gather(3)
        wait_gather(2)
        add(2)
        issue_out(c0 + 2, 2)
        wait_gather(3)
        add(3)
        issue_out(c0 + 3, 3)
        for b in range(NB_):
            wait_out(b)

    out128 = emb_kernel(x.astype(jnp.int32).reshape(-1), token_table, pos_table)
    return out128[:, :, :EMBED_]
